# baseline (speedup 1.0000x reference)
import functools

import jax
import jax.numpy as jnp
from jax import lax
from jax.experimental import pallas as pl
from jax.experimental.pallas import tpu as pltpu
from jax.experimental.pallas import tpu_sc as plsc

_SCALE = 8.0
_MARGIN = 0.2
_BS = 1024
_NCLS = 100000

_RB = 32
_GRID = _BS // _RB
_BC = 100352
_SLICE = 2048
_C = (_SCALE / 2.0) * 1.4426950408889634
_C2 = 0.7981034820053446


def _dense_body(e_ref, lbl_ref, s2_ref, w2_ref):
    t = jnp.right_shift(lbl_ref[...], 7)
    s2 = jnp.zeros((_RB, 1), jnp.float32)
    wraw = jnp.zeros((_RB, 128), jnp.float32)
    for j in range(_BC // _SLICE):
        base = j * _SLICE
        xs = e_ref[:, base:base + _SLICE]
        col = base + lax.broadcasted_iota(jnp.int32, xs.shape, 1)
        zs = jnp.exp2(_C * jnp.tanh(0.5 * xs))
        zs = jnp.where(col < _NCLS, zs, 0.0)
        s2 = s2 + jnp.sum(zs, axis=1, keepdims=True)
        sel = jnp.where(jnp.right_shift(col, 7) == t, xs, 0.0)
        for q in range(_SLICE // 128):
            wraw = wraw + sel[:, q * 128:(q + 1) * 128]
    s2_ref[...] = s2
    w2_ref[...] = jnp.exp2(_C * jnp.tanh(0.5 * wraw))


def _dense(e, lbl2d):
    return pl.pallas_call(
        _dense_body,
        grid=(_GRID,),
        in_specs=[
            pl.BlockSpec((_RB, _BC), lambda i: (i, 0)),
            pl.BlockSpec((_RB, 1), lambda i: (i, 0)),
        ],
        out_specs=[
            pl.BlockSpec((_RB, 1), lambda i: (i, 0)),
            pl.BlockSpec((_RB, 128), lambda i: (i, 0)),
        ],
        out_shape=[
            jax.ShapeDtypeStruct((_BS, 1), jnp.float32),
            jax.ShapeDtypeStruct((_BS, 128), jnp.float32),
        ],
    )(e, lbl2d)


def _comb_body(s2_ref, zl_ref, out_ref):
    s2 = s2_ref[...]
    zl = zl_ref[...]
    per = jnp.log(s2 - _C2 * zl) - jnp.log(zl)
    out_ref[...] = (jnp.sum(per, axis=0, keepdims=True) * (1.0 / _BS)
                    + _SCALE * _MARGIN)


def _combine(s2, zl):
    return pl.pallas_call(
        _comb_body,
        grid=(1,),
        in_specs=[
            pl.BlockSpec((_BS, 1), lambda i: (0, 0)),
            pl.BlockSpec((_BS, 1), lambda i: (0, 0)),
        ],
        out_specs=pl.BlockSpec((1, 1), lambda i: (0, 0)),
        out_shape=jax.ShapeDtypeStruct((1, 1), jnp.float32),
    )(s2, zl)


def _sc_gather(w2, label):
    info = plsc.get_sparse_core_info()
    nc, ns, nl = info.num_cores, info.num_subcores, info.num_lanes
    nw = nc * ns
    bpw = _BS // nw
    mesh = plsc.VectorSubcoreMesh(core_axis_name="c", subcore_axis_name="s")

    @functools.partial(
        pl.kernel,
        mesh=mesh,
        out_type=jax.ShapeDtypeStruct((_BS,), jnp.float32),
        scratch_types=[
            pltpu.VMEM((bpw,), jnp.int32),
            pltpu.VMEM((bpw, 128), jnp.float32),
            pltpu.VMEM((bpw,), jnp.float32),
        ],
        compiler_params=pltpu.CompilerParams(
            use_tc_tiling_on_sc=True, needs_layout_passes=False),
    )
    def k(w2_hbm, lbl_hbm, out_hbm, lbl_v, win_v, out_v):
        wid = lax.axis_index("s") * nc + lax.axis_index("c")
        base = wid * bpw
        pltpu.sync_copy(lbl_hbm.at[pl.ds(base, bpw)], lbl_v)
        pltpu.sync_copy(w2_hbm.at[pl.ds(base, bpw), :], win_v)
        for j in range(bpw // nl):
            lbl = lbl_v[pl.ds(j * nl, nl)]
            ridx = j * nl + lax.iota(jnp.int32, nl)
            off = jnp.bitwise_and(lbl, 127)
            out_v[pl.ds(j * nl, nl)] = plsc.load_gather(win_v, [ridx, off])
        pltpu.sync_copy(out_v, out_hbm.at[pl.ds(base, bpw)])

    return k(w2, label)


def kernel(embeddings, label):
    lbl = label.astype(jnp.int32)
    s2, w2 = _dense(embeddings, lbl.reshape(_BS, 1))
    zl = _sc_gather(w2, lbl)
    loss = _combine(s2, zl.reshape(_BS, 1))
    return loss[0, 0]


# speedup vs baseline: 1.0010x; 1.0010x over previous
import functools

import jax
import jax.numpy as jnp
from jax import lax
from jax.experimental import pallas as pl
from jax.experimental.pallas import tpu as pltpu
from jax.experimental.pallas import tpu_sc as plsc

_SCALE = 8.0
_MARGIN = 0.2
_BS = 1024
_NCLS = 100000

_RB = 32
_GRID = _BS // _RB
_NTILE = _NCLS // 128
_FULL = _NTILE * 128
_SLICE = 2048
_SLICES = [(j * _SLICE, _SLICE) for j in range(_FULL // _SLICE)]
if _FULL % _SLICE:
    _SLICES.append((_FULL - _FULL % _SLICE, _FULL % _SLICE))
_C = (_SCALE / 2.0) * 1.4426950408889634
_C2 = 0.7981034820053446


def _dense_body(e_ref, lbl_ref, s2_ref, w2_ref):
    t = jnp.right_shift(lbl_ref[...], 7)
    s2 = jnp.zeros((_RB, 1), jnp.float32)
    wraw = jnp.zeros((_RB, 128), jnp.float32)
    for start, width in _SLICES:
        xs = e_ref[:, start:start + width]
        tid = (start // 128) + jnp.right_shift(
            lax.broadcasted_iota(jnp.int32, xs.shape, 1), 7)
        zs = jnp.exp2(_C * jnp.tanh(0.5 * xs))
        s2 = s2 + jnp.sum(zs, axis=1, keepdims=True)
        sel = jnp.where(tid == t, xs, 0.0)
        for q in range(width // 128):
            wraw = wraw + sel[:, q * 128:(q + 1) * 128]
    xs_t = e_ref[:, _FULL:_NCLS]
    zt = jnp.exp2(_C * jnp.tanh(0.5 * xs_t))
    s2 = s2 + jnp.sum(zt, axis=1, keepdims=True)
    sel_t = jnp.where(t == _NTILE, xs_t, 0.0)
    wraw = wraw + jnp.concatenate(
        [sel_t, jnp.zeros((_RB, 128 - (_NCLS - _FULL)), jnp.float32)], axis=1)
    s2_ref[...] = s2
    w2_ref[...] = jnp.exp2(_C * jnp.tanh(0.5 * wraw))


def _dense(e, lbl2d):
    return pl.pallas_call(
        _dense_body,
        grid=(_GRID,),
        in_specs=[
            pl.BlockSpec((_RB, _NCLS), lambda i: (i, 0)),
            pl.BlockSpec((_RB, 1), lambda i: (i, 0)),
        ],
        out_specs=[
            pl.BlockSpec((_RB, 1), lambda i: (i, 0)),
            pl.BlockSpec((_RB, 128), lambda i: (i, 0)),
        ],
        out_shape=[
            jax.ShapeDtypeStruct((_BS, 1), jnp.float32),
            jax.ShapeDtypeStruct((_BS, 128), jnp.float32),
        ],
    )(e, lbl2d)


def _comb_body(s2_ref, zl_ref, out_ref):
    s2 = s2_ref[...]
    zl = zl_ref[...]
    per = jnp.log(s2 - _C2 * zl) - jnp.log(zl)
    out_ref[...] = (jnp.sum(per, axis=0, keepdims=True) * (1.0 / _BS)
                    + _SCALE * _MARGIN)


def _combine(s2, zl):
    return pl.pallas_call(
        _comb_body,
        grid=(1,),
        in_specs=[
            pl.BlockSpec((_BS, 1), lambda i: (0, 0)),
            pl.BlockSpec((_BS, 1), lambda i: (0, 0)),
        ],
        out_specs=pl.BlockSpec((1, 1), lambda i: (0, 0)),
        out_shape=jax.ShapeDtypeStruct((1, 1), jnp.float32),
    )(s2, zl)


def _sc_gather(w2, label):
    info = plsc.get_sparse_core_info()
    nc, ns, nl = info.num_cores, info.num_subcores, info.num_lanes
    nw = nc * ns
    bpw = _BS // nw
    mesh = plsc.VectorSubcoreMesh(core_axis_name="c", subcore_axis_name="s")

    @functools.partial(
        pl.kernel,
        mesh=mesh,
        out_type=jax.ShapeDtypeStruct((_BS,), jnp.float32),
        scratch_types=[
            pltpu.VMEM((bpw,), jnp.int32),
            pltpu.VMEM((bpw, 128), jnp.float32),
            pltpu.VMEM((bpw,), jnp.float32),
        ],
        compiler_params=pltpu.CompilerParams(
            use_tc_tiling_on_sc=True, needs_layout_passes=False),
    )
    def k(w2_hbm, lbl_hbm, out_hbm, lbl_v, win_v, out_v):
        wid = lax.axis_index("s") * nc + lax.axis_index("c")
        base = wid * bpw
        pltpu.sync_copy(lbl_hbm.at[pl.ds(base, bpw)], lbl_v)
        pltpu.sync_copy(w2_hbm.at[pl.ds(base, bpw), :], win_v)
        for j in range(bpw // nl):
            lbl = lbl_v[pl.ds(j * nl, nl)]
            ridx = j * nl + lax.iota(jnp.int32, nl)
            off = jnp.bitwise_and(lbl, 127)
            out_v[pl.ds(j * nl, nl)] = plsc.load_gather(win_v, [ridx, off])
        pltpu.sync_copy(out_v, out_hbm.at[pl.ds(base, bpw)])

    return k(w2, label)


def kernel(embeddings, label):
    lbl = label.astype(jnp.int32)
    s2, w2 = _dense(embeddings, lbl.reshape(_BS, 1))
    zl = _sc_gather(w2, lbl)
    loss = _combine(s2, zl.reshape(_BS, 1))
    return loss[0, 0]


# speedup vs baseline: 3.4287x; 3.4253x over previous
import functools

import jax
import jax.numpy as jnp
from jax import lax
from jax.experimental import pallas as pl
from jax.experimental.pallas import tpu as pltpu
from jax.experimental.pallas import tpu_sc as plsc

_SCALE = 8.0
_MARGIN = 0.2
_BS = 1024
_NCLS = 100000

_CB = 4000
_GRID = _NCLS // _CB
_C = (_SCALE / 2.0) * 1.4426950408889634
_E4 = 54.598150033144236


def _dense_body(et_ref, out_ref, acc_ref):
    i = pl.program_id(0)

    @pl.when(i == 0)
    def _():
        acc_ref[...] = jnp.zeros_like(acc_ref)

    z = jnp.exp2(_C * jnp.tanh(0.5 * et_ref[...]))
    acc_ref[...] += jnp.sum(z, axis=0, keepdims=True)

    @pl.when(i == _GRID - 1)
    def _():
        out_ref[...] = acc_ref[...]


def _dense(et):
    return pl.pallas_call(
        _dense_body,
        grid=(_GRID,),
        in_specs=[pl.BlockSpec((_CB, _BS), lambda i: (i, 0))],
        out_specs=pl.BlockSpec((1, _BS), lambda i: (0, 0)),
        out_shape=jax.ShapeDtypeStruct((1, _BS), jnp.float32),
        scratch_shapes=[pltpu.VMEM((1, _BS), jnp.float32)],
    )(et)


def _comb_body(s2_ref, g_ref, out_ref):
    s2 = s2_ref[...]
    g = g_ref[...]
    sig = 0.5 + 0.5 * jnp.tanh(0.5 * g)
    t = _SCALE * sig - _SCALE * _MARGIN
    sp = _E4 * s2 - jnp.exp(_SCALE * sig) + jnp.exp(t)
    per = jnp.log(sp) - t
    out_ref[...] = jnp.sum(per, axis=1, keepdims=True) * (1.0 / _BS)


def _combine(s2, g):
    return pl.pallas_call(
        _comb_body,
        grid=(1,),
        in_specs=[
            pl.BlockSpec((1, _BS), lambda i: (0, 0)),
            pl.BlockSpec((1, _BS), lambda i: (0, 0)),
        ],
        out_specs=pl.BlockSpec((1, 1), lambda i: (0, 0)),
        out_shape=jax.ShapeDtypeStruct((1, 1), jnp.float32),
    )(s2, g)


def _sc_gather(et, label):
    info = plsc.get_sparse_core_info()
    nc, ns, nl = info.num_cores, info.num_subcores, info.num_lanes
    nw = nc * ns
    bpw = _BS // nw
    mesh = plsc.VectorSubcoreMesh(core_axis_name="c", subcore_axis_name="s")

    @functools.partial(
        pl.kernel,
        mesh=mesh,
        out_type=jax.ShapeDtypeStruct((_BS,), jnp.float32),
        scratch_types=[
            pltpu.VMEM((bpw,), jnp.int32),
            pltpu.VMEM((bpw, _BS), jnp.float32),
            pltpu.VMEM((bpw,), jnp.float32),
            pltpu.SemaphoreType.DMA,
        ],
        compiler_params=pltpu.CompilerParams(
            use_tc_tiling_on_sc=True, needs_layout_passes=False),
    )
    def k(et_hbm, lbl_hbm, out_hbm, lbl_v, rows_v, out_v, sem):
        wid = lax.axis_index("s") * nc + lax.axis_index("c")
        base = wid * bpw
        pltpu.sync_copy(lbl_hbm.at[pl.ds(base, bpw)], lbl_v)
        pltpu.async_copy(et_hbm.at[lbl_v], rows_v, sem).wait()
        for j in range(bpw // nl):
            ridx = j * nl + lax.iota(jnp.int32, nl)
            out_v[pl.ds(j * nl, nl)] = plsc.load_gather(rows_v, [ridx, base + ridx])
        pltpu.sync_copy(out_v, out_hbm.at[pl.ds(base, bpw)])

    return k(et, label)


def kernel(embeddings, label):
    et = embeddings.T
    g = _sc_gather(et, label.astype(jnp.int32))
    s2 = _dense(et)
    loss = _combine(s2, g.reshape(1, _BS))
    return loss[0, 0]
